# baseline (speedup 1.0000x reference)
import functools

import jax
import jax.numpy as jnp
import numpy as np
from jax import lax
from jax.experimental import pallas as pl
from jax.experimental.pallas import tpu as pltpu
from jax.experimental.pallas import tpu_sc as plsc

B, N, HID, H, NREL, EPN = 4, 128, 768, 12, 64, 32
E = B * N * EPN
DH = HID // H
LH = 16
NT = 16
EPT = E // NT
NCH = EPT // 128
ROWS_T = B * N * N
ROWS_G = B * N * NREL
f32 = jnp.float32


def _proj_body(ns_ref, wqt_ref, wkt_ref, wvt_ref, bq_ref, bk_ref, bv_ref,
               mk_ref, q_ref, k_ref, v_ref, qe_ref):
    ns = ns_ref[...]
    q = jnp.dot(ns, wqt_ref[...], preferred_element_type=f32) + bq_ref[...]
    k = jnp.dot(ns, wkt_ref[...], preferred_element_type=f32) + bk_ref[...]
    v = jnp.dot(ns, wvt_ref[...], preferred_element_type=f32) + bv_ref[...]
    q_ref[...] = q
    k_ref[...] = k
    v_ref[...] = v
    qe = jnp.dot(q, mk_ref[...], preferred_element_type=f32)
    lane = lax.broadcasted_iota(jnp.int32, (B * N, NREL * LH), 1)
    qe_ref[...] = qe + jnp.where(lane % LH == 12, 1.0, 0.0).astype(f32)


_proj_call = pl.pallas_call(
    _proj_body,
    out_shape=(
        jax.ShapeDtypeStruct((B * N, HID), f32),
        jax.ShapeDtypeStruct((B * N, HID), f32),
        jax.ShapeDtypeStruct((B * N, HID), f32),
        jax.ShapeDtypeStruct((B * N, NREL * LH), f32),
    ),
)


NW = 16
RPW = B * N // NW
EBLK = 2048
NBLK = E // EBLK


@functools.lru_cache(maxsize=None)
def _sc_scatter_kernel(src_per_row, acc_per_row, nlane, gather_by_g):
    spt = RPW * src_per_row
    zpt = RPW * acc_per_row

    def body(src_hbm, bv_hbm, iv_hbm, jv_hbm, rv_hbm, z_hbm, out_hbm,
             bb, ib, jb, rb, cg, ct, src_v, acc_v):
        cid = lax.axis_index("c")
        sid = lax.axis_index("s")
        wid = sid
        lo = wid * RPW
        lanes = [jnp.full((16,), l, jnp.int32) for l in range(nlane)]
        liota = lax.iota(jnp.int32, 16)

        @pl.when(cid == 0)
        def _():
          pltpu.sync_copy(src_hbm.at[pl.ds(lo * src_per_row, spt)], src_v)
          pltpu.sync_copy(z_hbm.at[pl.ds(0, zpt)], acc_v)

          def scan_blk(blk, carry):
            base = blk * EBLK
            pltpu.sync_copy(bv_hbm.at[pl.ds(base, EBLK)], bb)
            pltpu.sync_copy(iv_hbm.at[pl.ds(base, EBLK)], ib)
            pltpu.sync_copy(jv_hbm.at[pl.ds(base, EBLK)], jb)
            pltpu.sync_copy(rv_hbm.at[pl.ds(base, EBLK)], rb)

            def grp(i, cnt):
                off = i * 16
                bv = bb[pl.ds(off, 16)]
                iv = ib[pl.ds(off, 16)]
                jv = jb[pl.ds(off, 16)]
                rv = rb[pl.ds(off, 16)]
                rowv = bv * N + iv
                inr = (rowv >= lo) & (rowv < lo + RPW)
                rloc = rowv - lo
                gl = rloc * NREL + rv
                tl = rloc * N + jv
                gidx = gl if gather_by_g else tl
                sidx = tl if gather_by_g else gl
                plsc.store_compressed(cg.at[pl.ds(cnt, 16)], gidx, mask=inr)
                plsc.store_compressed(ct.at[pl.ds(cnt, 16)], sidx, mask=inr)
                return cnt + plsc.all_reduce_population_count(inr)[0]

            cnt = lax.fori_loop(0, EBLK // 16, grp, jnp.int32(0))

            def proc(i, carry2):
                off = i * 16
                valid = off + liota < cnt
                gidx = cg[pl.ds(off, 16)]
                sidx = ct[pl.ds(off, 16)]
                gidx = jnp.minimum(jnp.maximum(gidx, 0), spt - 1)
                sidx = jnp.minimum(jnp.maximum(sidx, 0), zpt - 1)
                for l in range(nlane):
                    vals = plsc.load_gather(src_v, [gidx, lanes[l]], mask=valid)
                    plsc.addupdate_scatter(acc_v, [sidx, lanes[l]], vals,
                                           mask=valid)
                return carry2

            lax.fori_loop(0, (cnt + 15) // 16, proc, jnp.int32(0))
            return carry

          lax.fori_loop(0, NBLK, scan_blk, jnp.int32(0))
          pltpu.sync_copy(acc_v, out_hbm.at[pl.ds(lo * acc_per_row, zpt)])

    return pl.kernel(
        body,
        out_type=jax.ShapeDtypeStruct((NW * zpt, LH), f32),
        mesh=plsc.VectorSubcoreMesh(core_axis_name="c", subcore_axis_name="s",
                                    num_cores=2, num_subcores=NT),
        compiler_params=pltpu.CompilerParams(needs_layout_passes=False,
                                             use_tc_tiling_on_sc=False),
        scratch_types=[
            pltpu.VMEM((EBLK,), jnp.int32),
            pltpu.VMEM((EBLK,), jnp.int32),
            pltpu.VMEM((EBLK,), jnp.int32),
            pltpu.VMEM((EBLK,), jnp.int32),
            pltpu.VMEM((EBLK + 16,), jnp.int32),
            pltpu.VMEM((EBLK + 16,), jnp.int32),
            pltpu.VMEM((spt, LH), f32),
            pltpu.VMEM((zpt, LH), f32),
        ],
    )


def _attn_body(q_ref, k_ref, v_ref, s2e_ref, cnt_ref, out1_ref, probs_ref):
    q = q_ref[0, 0]
    k = k_ref[0, 0]
    inv = np.float32(1.0 / np.sqrt(DH))
    n2n = lax.dot_general(q, k, (((1,), (1,)), ((), ())),
                          preferred_element_type=f32)
    logits = (n2n + s2e_ref[0, 0]) * inv
    mask = cnt_ref[0] > 0.5
    m = jnp.max(jnp.where(mask, logits, np.float32(-1e30)), axis=1, keepdims=True)
    m = jnp.where(m < -1e29, np.float32(0.0), m)
    ex = jnp.exp(logits - m) * mask.astype(f32)
    den = jnp.maximum(jnp.sum(ex, axis=1, keepdims=True), np.float32(1e-30))
    p = ex / den
    probs_ref[0, 0] = p
    out1_ref[0, 0] = jnp.dot(p, v_ref[0, 0], preferred_element_type=f32)


_attn_call = pl.pallas_call(
    _attn_body,
    grid=(B, H),
    in_specs=[
        pl.BlockSpec((1, 1, N, DH), lambda b, h: (b, h, 0, 0)),
        pl.BlockSpec((1, 1, N, DH), lambda b, h: (b, h, 0, 0)),
        pl.BlockSpec((1, 1, N, DH), lambda b, h: (b, h, 0, 0)),
        pl.BlockSpec((1, 1, N, N), lambda b, h: (b, h, 0, 0)),
        pl.BlockSpec((1, N, N), lambda b, h: (b, 0, 0)),
    ],
    out_specs=[
        pl.BlockSpec((1, 1, N, DH), lambda b, h: (b, h, 0, 0)),
        pl.BlockSpec((1, 1, N, N), lambda b, h: (b, h, 0, 0)),
    ],
    out_shape=(
        jax.ShapeDtypeStruct((B, H, N, DH), f32),
        jax.ShapeDtypeStruct((B, H, N, N), f32),
    ),
)


def _final_body(w_ref, mv_ref, out1_ref, out_ref):
    out_ref[...] = out1_ref[...] + jnp.dot(w_ref[...], mv_ref[...],
                                           preferred_element_type=f32)


_final_call = pl.pallas_call(
    _final_body,
    out_shape=jax.ShapeDtypeStruct((B * N, HID), f32),
)


def kernel(node_states, edge_indices, node_type_ids, Wq, bq, Wk, bk, Wv, bv,
           key_edge_table, value_edge_table):
    ns = node_states.reshape(B * N, HID)
    eye = jnp.eye(H, LH, dtype=f32)
    MK = jnp.einsum("rhd,hl->hdrl", key_edge_table.reshape(NREL, H, DH),
                    eye).reshape(HID, NREL * LH)
    MV = jnp.einsum("rhd,hl->rlhd", value_edge_table.reshape(NREL, H, DH),
                    eye).reshape(NREL * LH, HID)
    zeros_blk = jnp.zeros((ROWS_T // NT, LH), f32)

    q, k, v, qe = _proj_call(ns, Wq.T, Wk.T, Wv.T, bq[None, :], bk[None, :],
                             bv[None, :], MK)

    ei = edge_indices
    s2e = _sc_scatter_kernel(NREL, N, 13, True)(qe.reshape(ROWS_G, LH),
                                                ei[0], ei[1], ei[2], ei[3],
                                                zeros_blk)
    s2e_t = s2e.reshape(B, N, N, LH).transpose(0, 3, 1, 2)
    cnt = s2e_t[:, 12]

    qh = q.reshape(B, N, H, DH).transpose(0, 2, 1, 3)
    kh = k.reshape(B, N, H, DH).transpose(0, 2, 1, 3)
    vh = v.reshape(B, N, H, DH).transpose(0, 2, 1, 3)
    out1, probs = _attn_call(qh, kh, vh, s2e_t, cnt)

    probs_pad = jnp.pad(probs.transpose(0, 2, 3, 1),
                        ((0, 0), (0, 0), (0, 0), (0, LH - H)))
    w = _sc_scatter_kernel(N, NREL, 12, False)(probs_pad.reshape(ROWS_T, LH),
                                               ei[0], ei[1], ei[2], ei[3],
                                               zeros_blk)

    out1_flat = out1.transpose(0, 2, 1, 3).reshape(B * N, HID)
    out = _final_call(w.reshape(B * N, NREL * LH), MV, out1_flat)
    return out.reshape(B, N, HID)


# speedup vs baseline: 1.0898x; 1.0898x over previous
import functools

import jax
import jax.numpy as jnp
import numpy as np
from jax import lax
from jax.experimental import pallas as pl
from jax.experimental.pallas import tpu as pltpu
from jax.experimental.pallas import tpu_sc as plsc

B, N, HID, H, NREL, EPN = 4, 128, 768, 12, 64, 32
E = B * N * EPN
DH = HID // H
LH = 16
NT = 16
EPT = E // NT
NCH = EPT // 128
ROWS_T = B * N * N
ROWS_G = B * N * NREL
f32 = jnp.float32


def _proj_body(ns_ref, wqt_ref, wkt_ref, wvt_ref, bq_ref, bk_ref, bv_ref,
               mk_ref, eb_ref, ei_ref, ej_ref, er_ref,
               q_ref, k_ref, v_ref, qe_ref, g_ref, t_ref):
    ns = ns_ref[...]
    q = jnp.dot(ns, wqt_ref[...], preferred_element_type=f32) + bq_ref[...]
    k = jnp.dot(ns, wkt_ref[...], preferred_element_type=f32) + bk_ref[...]
    v = jnp.dot(ns, wvt_ref[...], preferred_element_type=f32) + bv_ref[...]
    q_ref[...] = q
    k_ref[...] = k
    v_ref[...] = v
    qe = jnp.dot(q, mk_ref[...], preferred_element_type=f32)
    lane = lax.broadcasted_iota(jnp.int32, (B * N, NREL * LH), 1)
    qe_ref[...] = qe + jnp.where(lane % LH == 12, 1.0, 0.0).astype(f32)
    row = eb_ref[...] * N + ei_ref[...]
    g_ref[...] = row * NREL + er_ref[...]
    t_ref[...] = row * N + ej_ref[...]


_proj_call = pl.pallas_call(
    _proj_body,
    out_shape=(
        jax.ShapeDtypeStruct((B * N, HID), f32),
        jax.ShapeDtypeStruct((B * N, HID), f32),
        jax.ShapeDtypeStruct((B * N, HID), f32),
        jax.ShapeDtypeStruct((B * N, NREL * LH), f32),
        jax.ShapeDtypeStruct((128, E // 128), jnp.int32),
        jax.ShapeDtypeStruct((128, E // 128), jnp.int32),
    ),
)


NW = 32
RPW = B * N // NW
EBLK = 2048
NBLK = E // EBLK


@functools.lru_cache(maxsize=None)
def _sc_scatter_kernel(src_per_row, acc_per_row, nlane, gather_by_g):
    spt = RPW * src_per_row
    zpt = RPW * acc_per_row

    gshift = 6 if gather_by_g else 7

    def body(src_hbm, gi_hbm, ti_hbm, z_hbm, out_hbm,
             gb, tb, cg, ct, src_v, acc_v):
        cid = lax.axis_index("c")
        sid = lax.axis_index("s")
        wid = sid * 2 + cid
        lo = wid * RPW
        gbase = lo * src_per_row
        sbase = lo * acc_per_row
        lanes = [jnp.full((16,), l, jnp.int32) for l in range(nlane)]
        liota = lax.iota(jnp.int32, 16)
        pltpu.sync_copy(src_hbm.at[pl.ds(gbase, spt)], src_v)
        pltpu.sync_copy(z_hbm.at[pl.ds(0, zpt)], acc_v)

        def scan_blk(blk, carry):
            base = blk * EBLK
            pltpu.sync_copy(gi_hbm.at[pl.ds(base, EBLK)], gb)
            pltpu.sync_copy(ti_hbm.at[pl.ds(base, EBLK)], tb)

            def grp(i, cnt):
                off = i * 16
                gv = gb[pl.ds(off, 16)]
                tv = tb[pl.ds(off, 16)]
                rowv = lax.shift_right_logical(gv, gshift)
                inr = (rowv >= lo) & (rowv < lo + RPW)
                plsc.store_compressed(cg.at[pl.ds(cnt, 16)], gv, mask=inr)
                plsc.store_compressed(ct.at[pl.ds(cnt, 16)], tv, mask=inr)
                return cnt + plsc.all_reduce_population_count(inr)[0]

            cnt = lax.fori_loop(0, EBLK // 16, grp, jnp.int32(0))

            def proc(i, carry2):
                off = i * 16
                valid = off + liota < cnt
                gidx = cg[pl.ds(off, 16)] - gbase
                sidx = ct[pl.ds(off, 16)] - sbase
                gidx = jnp.minimum(jnp.maximum(gidx, 0), spt - 1)
                sidx = jnp.minimum(jnp.maximum(sidx, 0), zpt - 1)
                for l in range(nlane):
                    vals = plsc.load_gather(src_v, [gidx, lanes[l]], mask=valid)
                    plsc.addupdate_scatter(acc_v, [sidx, lanes[l]], vals,
                                           mask=valid)
                return carry2

            lax.fori_loop(0, (cnt + 15) // 16, proc, jnp.int32(0))
            return carry

        lax.fori_loop(0, NBLK, scan_blk, jnp.int32(0))
        pltpu.sync_copy(acc_v, out_hbm.at[pl.ds(sbase, zpt)])

    return pl.kernel(
        body,
        out_type=jax.ShapeDtypeStruct((NW * zpt, LH), f32),
        mesh=plsc.VectorSubcoreMesh(core_axis_name="c", subcore_axis_name="s",
                                    num_cores=2, num_subcores=NT),
        compiler_params=pltpu.CompilerParams(needs_layout_passes=False,
                                             use_tc_tiling_on_sc=False),
        scratch_types=[
            pltpu.VMEM((EBLK,), jnp.int32),
            pltpu.VMEM((EBLK,), jnp.int32),
            pltpu.VMEM((EBLK + 16,), jnp.int32),
            pltpu.VMEM((EBLK + 16,), jnp.int32),
            pltpu.VMEM((spt, LH), f32),
            pltpu.VMEM((zpt, LH), f32),
        ],
    )


def _attn_body(q_ref, k_ref, v_ref, s2e_ref, cnt_ref, out1_ref, probs_ref):
    q = q_ref[0, 0]
    k = k_ref[0, 0]
    inv = np.float32(1.0 / np.sqrt(DH))
    n2n = lax.dot_general(q, k, (((1,), (1,)), ((), ())),
                          preferred_element_type=f32)
    logits = (n2n + s2e_ref[0, 0]) * inv
    mask = cnt_ref[0] > 0.5
    m = jnp.max(jnp.where(mask, logits, np.float32(-1e30)), axis=1, keepdims=True)
    m = jnp.where(m < -1e29, np.float32(0.0), m)
    ex = jnp.exp(logits - m) * mask.astype(f32)
    den = jnp.maximum(jnp.sum(ex, axis=1, keepdims=True), np.float32(1e-30))
    p = ex / den
    probs_ref[0, 0] = p
    out1_ref[0, 0] = jnp.dot(p, v_ref[0, 0], preferred_element_type=f32)


_attn_call = pl.pallas_call(
    _attn_body,
    grid=(B, H),
    in_specs=[
        pl.BlockSpec((1, 1, N, DH), lambda b, h: (b, h, 0, 0)),
        pl.BlockSpec((1, 1, N, DH), lambda b, h: (b, h, 0, 0)),
        pl.BlockSpec((1, 1, N, DH), lambda b, h: (b, h, 0, 0)),
        pl.BlockSpec((1, 1, N, N), lambda b, h: (b, h, 0, 0)),
        pl.BlockSpec((1, N, N), lambda b, h: (b, 0, 0)),
    ],
    out_specs=[
        pl.BlockSpec((1, 1, N, DH), lambda b, h: (b, h, 0, 0)),
        pl.BlockSpec((1, 1, N, N), lambda b, h: (b, h, 0, 0)),
    ],
    out_shape=(
        jax.ShapeDtypeStruct((B, H, N, DH), f32),
        jax.ShapeDtypeStruct((B, H, N, N), f32),
    ),
)


def _final_body(w_ref, mv_ref, out1_ref, out_ref):
    out_ref[...] = out1_ref[...] + jnp.dot(w_ref[...], mv_ref[...],
                                           preferred_element_type=f32)


_final_call = pl.pallas_call(
    _final_body,
    out_shape=jax.ShapeDtypeStruct((B * N, HID), f32),
)


def kernel(node_states, edge_indices, node_type_ids, Wq, bq, Wk, bk, Wv, bv,
           key_edge_table, value_edge_table):
    ns = node_states.reshape(B * N, HID)
    eye = jnp.eye(H, LH, dtype=f32)
    MK = jnp.einsum("rhd,hl->hdrl", key_edge_table.reshape(NREL, H, DH),
                    eye).reshape(HID, NREL * LH)
    MV = jnp.einsum("rhd,hl->rlhd", value_edge_table.reshape(NREL, H, DH),
                    eye).reshape(NREL * LH, HID)
    zeros_blk = jnp.zeros((ROWS_T // NT, LH), f32)

    ei2d = edge_indices.reshape(4, 128, E // 128)
    q, k, v, qe, g2d, t2d = _proj_call(ns, Wq.T, Wk.T, Wv.T, bq[None, :],
                                       bk[None, :], bv[None, :], MK,
                                       ei2d[0], ei2d[1], ei2d[2], ei2d[3])
    g_idx = g2d.reshape(E)
    t_idx = t2d.reshape(E)

    s2e = _sc_scatter_kernel(NREL, N, 13, True)(qe.reshape(ROWS_G, LH),
                                                g_idx, t_idx, zeros_blk)
    s2e_t = s2e.reshape(B, N, N, LH).transpose(0, 3, 1, 2)
    cnt = s2e_t[:, 12]

    qh = q.reshape(B, N, H, DH).transpose(0, 2, 1, 3)
    kh = k.reshape(B, N, H, DH).transpose(0, 2, 1, 3)
    vh = v.reshape(B, N, H, DH).transpose(0, 2, 1, 3)
    out1, probs = _attn_call(qh, kh, vh, s2e_t, cnt)

    probs_pad = jnp.pad(probs.transpose(0, 2, 3, 1),
                        ((0, 0), (0, 0), (0, 0), (0, LH - H)))
    w = _sc_scatter_kernel(N, NREL, 12, False)(probs_pad.reshape(ROWS_T, LH),
                                               t_idx, g_idx, zeros_blk)

    out1_flat = out1.transpose(0, 2, 1, 3).reshape(B * N, HID)
    out = _final_call(w.reshape(B * N, NREL * LH), MV, out1_flat)
    return out.reshape(B, N, HID)
